# baseline (speedup 1.0000x reference)
import functools

import jax
import jax.numpy as jnp
from jax import lax
from jax.experimental import pallas as pl
from jax.experimental.pallas import tpu as pltpu
from jax.experimental.pallas import tpu_sc as plsc

TOP_K = 8
N_EMBD = 4096
N_EXPERTS = 64
TOKENS = 8192
BT = 512

_INFO = plsc.get_sparse_core_info()
NC, NS, L = _INFO.num_cores, _INFO.num_subcores, _INFO.num_lanes
NW = NC * NS
TPW = TOKENS // NW


def _logits_body(x_ref, w_ref, out_ref):
    x = x_ref[...]
    w = w_ref[...]
    out_ref[...] = lax.dot_general(
        x, w, (((1,), (1,)), ((), ())), preferred_element_type=jnp.float32
    )


def _tc_logits(x, W):
    return pl.pallas_call(
        _logits_body,
        grid=(TOKENS // BT,),
        in_specs=[
            pl.BlockSpec((BT, N_EMBD), lambda i: (i, 0)),
            pl.BlockSpec((N_EXPERTS, N_EMBD), lambda i: (0, 0)),
        ],
        out_specs=pl.BlockSpec((BT, N_EXPERTS), lambda i: (i, 0)),
        out_shape=jax.ShapeDtypeStruct((TOKENS, N_EXPERTS), jnp.float32),
    )(x, W)


_MESH = plsc.VectorSubcoreMesh(core_axis_name="c", subcore_axis_name="s")


@functools.partial(
    pl.kernel,
    mesh=_MESH,
    out_type=[
        jax.ShapeDtypeStruct((TOKENS * TOP_K,), jnp.float32),
        jax.ShapeDtypeStruct((TOKENS * TOP_K,), jnp.int32),
    ],
    scratch_types=[
        pltpu.VMEM((TPW * N_EXPERTS,), jnp.float32),
        pltpu.VMEM((TPW * TOP_K + 2 * L,), jnp.float32),
        pltpu.VMEM((TPW * TOP_K + 2 * L,), jnp.int32),
    ],
    compiler_params=pltpu.CompilerParams(needs_layout_passes=False),
)
def _sc_topk(logits_hbm, w_out, i_out, lg_v, wv, iv):
    wid = lax.axis_index("s") * NC + lax.axis_index("c")
    base = wid * TPW * N_EXPERTS
    pltpu.sync_copy(logits_hbm.at[pl.ds(base, TPW * N_EXPERTS)], lg_v)

    lane = lax.broadcasted_iota(jnp.int32, (L,), 0)
    low8 = lane < TOP_K
    idx_consts = [lane + e * L for e in range(N_EXPERTS // L)]

    def merge(ka, va, kb, vb):
        km = jnp.where(low8, ka, lax.rev(kb, (0,)))
        vm = jnp.where(low8, va, lax.rev(vb, (0,)))
        return plsc.sort_key_val(km, vm, descending=True)

    def body(t, _):
        off = t * N_EXPERTS
        ks, vs = [], []
        for e in range(N_EXPERTS // L):
            k = lg_v[pl.ds(off + e * L, L)]
            sk, sv = plsc.sort_key_val(k, idx_consts[e], descending=True)
            ks.append(sk)
            vs.append(sv)
        k01, v01 = merge(ks[0], vs[0], ks[1], vs[1])
        k23, v23 = merge(ks[2], vs[2], ks[3], vs[3])
        kf, vf = merge(k01, v01, k23, v23)

        m = lax.reduce_max(kf, axes=(0,))
        e8 = jnp.where(low8, jnp.exp(kf - m), 0.0)
        s = lax.reduce_sum(e8, axes=(0,))
        w = e8 / s

        wv[pl.ds(t * TOP_K, L)] = w
        iv[pl.ds(t * TOP_K, L)] = vf
        return _

    lax.fori_loop(0, TPW, body, None)

    obase = wid * TPW * TOP_K
    pltpu.sync_copy(wv.at[pl.ds(0, TPW * TOP_K)], w_out.at[pl.ds(obase, TPW * TOP_K)])
    pltpu.sync_copy(iv.at[pl.ds(0, TPW * TOP_K)], i_out.at[pl.ds(obase, TPW * TOP_K)])


@jax.jit
def kernel(x, W):
    logits = _tc_logits(x, W).reshape(TOKENS * N_EXPERTS)
    w_flat, i_flat = _sc_topk(logits)
    return w_flat.reshape(TOKENS, TOP_K), i_flat.reshape(TOKENS, TOP_K)
